# Initial kernel scaffold; baseline (speedup 1.0000x reference)
import jax
import jax.numpy as jnp
from jax.experimental import pallas as pl


def kernel(x, edge_index, batch, W0, b0, g0, be0, W1, b1, g1, be1, W2, b2, g2, be2):
    raise NotImplementedError("write your pallas kernel here")


# speedup vs baseline: 8.7849x; 8.7849x over previous
import functools

import jax
import jax.numpy as jnp
from jax import lax
from jax.experimental import pallas as pl
from jax.experimental.pallas import tpu as pltpu
from jax.experimental.pallas import tpu_sc as plsc

EPS = 1e-5
NC = 2
NS = 16
LANES = 16
CH = 128


def _sc_mesh():
    return plsc.VectorSubcoreMesh(core_axis_name="c", subcore_axis_name="s",
                                  num_cores=NC)


def _make_deg_kernel(NPAD, EPT):
    NCHUNK = EPT // CH
    RPT = NPAD // NS
    NZC = RPT // CH

    @functools.partial(
        pl.kernel,
        out_type=jax.ShapeDtypeStruct((NC, NPAD, LANES), jnp.float32),
        mesh=_sc_mesh(),
        scratch_types=[
            pltpu.VMEM((CH,), jnp.int32),
            pltpu.VMEM((CH, LANES), jnp.float32),
            pltpu.VMEM((CH, LANES), jnp.float32),
            pltpu.VMEM_SHARED((NPAD, LANES), jnp.float32),
        ],
    )
    def deg_kernel(dst_hbm, out_hbm, didx, ones, zbuf, acc):
        c = lax.axis_index("c")
        s = lax.axis_index("s")
        wid = c * NS + s

        lane = lax.iota(jnp.int32, 16)
        onev = jnp.where(lane == 0, 1.0, 0.0).astype(jnp.float32)
        zv = jnp.zeros((16,), jnp.float32)

        def fill(i, carry):
            ones[i, pl.ds(0, 16)] = onev
            zbuf[i, pl.ds(0, 16)] = zv
            return carry
        lax.fori_loop(0, CH, fill, 0)

        def zacc(i, carry):
            pltpu.sync_copy(zbuf, acc.at[pl.ds(s * RPT + i * CH, CH)])
            return carry
        lax.fori_loop(0, NZC, zacc, 0)
        plsc.subcore_barrier()

        base = wid * EPT

        def chunk(i, carry):
            pltpu.sync_copy(dst_hbm.at[pl.ds(base + i * CH, CH)], didx)
            pltpu.sync_copy(ones, acc.at[didx], add=True)
            return carry
        lax.fori_loop(0, NCHUNK, chunk, 0)
        plsc.subcore_barrier()

        pltpu.sync_copy(acc.at[pl.ds(s * RPT, RPT)],
                        out_hbm.at[c].at[pl.ds(s * RPT, RPT)])

    return deg_kernel


def _make_spmm_kernel(N, D, NPAD, EPT):
    NCHUNK = EPT // CH
    RPT = NPAD // NS
    NZC = RPT // CH

    @functools.partial(
        pl.kernel,
        out_type=jax.ShapeDtypeStruct((NC, NPAD, D), jnp.float32),
        mesh=_sc_mesh(),
        scratch_types=[
            pltpu.VMEM((CH,), jnp.int32),
            pltpu.VMEM((CH,), jnp.int32),
            pltpu.VMEM((CH, D), jnp.float32),
            pltpu.VMEM_SHARED((NPAD, D), jnp.float32),
            pltpu.SemaphoreType.DMA,
        ],
    )
    def spmm_kernel(ys_hbm, src_hbm, dst_hbm, out_hbm, sidx, didx, rows, acc,
                    sem):
        c = lax.axis_index("c")
        s = lax.axis_index("s")
        wid = c * NS + s

        zv = jnp.zeros((16,), jnp.float32)

        def zrow(i, carry):
            rows[i // (D // 16), pl.ds((i % (D // 16)) * 16, 16)] = zv
            return carry
        lax.fori_loop(0, CH * D // 16, zrow, 0)

        def zacc(i, carry):
            pltpu.sync_copy(rows, acc.at[pl.ds(s * RPT + i * CH, CH)])
            return carry
        lax.fori_loop(0, NZC, zacc, 0)
        plsc.subcore_barrier()

        base = wid * EPT

        def chunk(i, carry):
            off = base + i * CH
            pltpu.sync_copy(src_hbm.at[pl.ds(off, CH)], sidx)
            pltpu.sync_copy(dst_hbm.at[pl.ds(off, CH)], didx)
            pltpu.async_copy(ys_hbm.at[sidx], rows, sem).wait()
            pltpu.sync_copy(rows, acc.at[didx], add=True)
            return carry
        lax.fori_loop(0, NCHUNK, chunk, 0)
        plsc.subcore_barrier()

        pltpu.sync_copy(acc.at[pl.ds(s * RPT, RPT)],
                        out_hbm.at[c].at[pl.ds(s * RPT, RPT)])

    return spmm_kernel


def _dinv_from(dg):
    return lax.rsqrt(dg[0, :, 0] + dg[1, :, 0] + 1.0)


def _pre_body(x_ref, w_ref, dg_ref, o_ref):
    dinv = _dinv_from(dg_ref[...])
    y = jnp.dot(x_ref[...], w_ref[...], preferred_element_type=jnp.float32)
    o_ref[...] = y * dinv[:, None]


def _sum_body(p_ref, ys_ref, dg_ref, z_ref, st_ref, *, n_rows):
    i = pl.program_id(0)
    dinv = _dinv_from(dg_ref[...])
    z = (p_ref[0] + p_ref[1] + ys_ref[...]) * dinv[:, None]
    z_ref[...] = z
    s1 = jnp.sum(z, axis=0)
    s2 = jnp.sum(z * z, axis=0)
    acc = jnp.concatenate(
        [s1[None, :], s2[None, :], jnp.zeros((6, z.shape[1]), jnp.float32)],
        axis=0)

    @pl.when(i == 0)
    def _():
        st_ref[...] = acc

    @pl.when(i > 0)
    def _():
        st_ref[...] = st_ref[...] + acc


def _bn_relu(z, st, gb, n_rows):
    inv_n = 1.0 / n_rows
    mu = st[0, :] * inv_n
    var = st[1, :] * inv_n - mu * mu
    rstd = lax.rsqrt(var + EPS)
    h = gb[0, :] * ((z - mu) * rstd) + gb[1, :]
    return jnp.maximum(h, 0.0)


def _apply_body(z_ref, st_ref, gb_ref, dg_ref, w_ref, o_ref, *, n_rows):
    h = _bn_relu(z_ref[...], st_ref[...], gb_ref[...], n_rows)
    dinv = _dinv_from(dg_ref[...])
    y = jnp.dot(h, w_ref[...], preferred_element_type=jnp.float32)
    o_ref[...] = y * dinv[:, None]


def _final_body(z_ref, st_ref, gb_ref, o_ref, *, n_rows):
    o_ref[...] = _bn_relu(z_ref[...], st_ref[...], gb_ref[...], n_rows)


def kernel(x, edge_index, batch, W0, b0, g0, be0, W1, b1, g1, be1,
           W2, b2, g2, be2):
    del batch, b0, b1, b2
    N, D = x.shape
    H = W0.shape[1]
    src, dst = edge_index[0], edge_index[1]
    E = src.shape[0]

    NW = NC * NS
    EPT = -(-E // (NW * CH)) * CH
    EPAD = NW * EPT
    NPAD = -(-(N + 1) // (NS * CH)) * (NS * CH)
    pad = EPAD - E
    src_p = jnp.concatenate([src, jnp.zeros((pad,), src.dtype)])
    dst_p = jnp.concatenate([dst, jnp.full((pad,), N, dst.dtype)])

    deg_k = _make_deg_kernel(NPAD, EPT)
    spmm_k = _make_spmm_kernel(N, D, NPAD, EPT)

    BM = 1000
    NB = N // BM
    f32 = jnp.float32

    degp = deg_k(dst_p)

    dg_spec = pl.BlockSpec((2, BM, LANES), lambda i: (0, i, 0))

    pre = pl.pallas_call(
        _pre_body,
        grid=(NB,),
        in_specs=[
            pl.BlockSpec((BM, D), lambda i: (i, 0)),
            pl.BlockSpec((D, H), lambda i: (0, 0)),
            dg_spec,
        ],
        out_specs=pl.BlockSpec((BM, H), lambda i: (i, 0)),
        out_shape=jax.ShapeDtypeStruct((N, H), f32),
    )

    sum_call = pl.pallas_call(
        functools.partial(_sum_body, n_rows=N),
        grid=(NB,),
        in_specs=[
            pl.BlockSpec((NC, BM, H), lambda i: (0, i, 0)),
            pl.BlockSpec((BM, H), lambda i: (i, 0)),
            dg_spec,
        ],
        out_specs=[
            pl.BlockSpec((BM, H), lambda i: (i, 0)),
            pl.BlockSpec((8, H), lambda i: (0, 0)),
        ],
        out_shape=[
            jax.ShapeDtypeStruct((N, H), f32),
            jax.ShapeDtypeStruct((8, H), f32),
        ],
    )

    def apply_call(Wn):
        return pl.pallas_call(
            functools.partial(_apply_body, n_rows=N),
            grid=(NB,),
            in_specs=[
                pl.BlockSpec((BM, H), lambda i: (i, 0)),
                pl.BlockSpec((8, H), lambda i: (0, 0)),
                pl.BlockSpec((8, H), lambda i: (0, 0)),
                dg_spec,
                pl.BlockSpec((H, H), lambda i: (0, 0)),
            ],
            out_specs=pl.BlockSpec((BM, H), lambda i: (i, 0)),
            out_shape=jax.ShapeDtypeStruct((N, H), f32),
        )

    final_call = pl.pallas_call(
        functools.partial(_final_body, n_rows=N),
        grid=(NB,),
        in_specs=[
            pl.BlockSpec((BM, H), lambda i: (i, 0)),
            pl.BlockSpec((8, H), lambda i: (0, 0)),
            pl.BlockSpec((8, H), lambda i: (0, 0)),
        ],
        out_specs=pl.BlockSpec((BM, H), lambda i: (i, 0)),
        out_shape=jax.ShapeDtypeStruct((N, H), f32),
    )

    def gb(g, be):
        z6 = jnp.zeros((6, H), f32)
        return jnp.concatenate([g[None, :], be[None, :], z6], axis=0)

    ys = pre(x, W0, degp)

    parts = spmm_k(ys, src_p, dst_p)
    z, st = sum_call(parts, ys, degp)
    ys = apply_call(W1)(z, st, gb(g0, be0), degp, W1)

    parts = spmm_k(ys, src_p, dst_p)
    z, st = sum_call(parts, ys, degp)
    ys = apply_call(W2)(z, st, gb(g1, be1), degp, W2)

    parts = spmm_k(ys, src_p, dst_p)
    z, st = sum_call(parts, ys, degp)
    return final_call(z, st, gb(g2, be2))
